# Initial kernel scaffold; baseline (speedup 1.0000x reference)
import jax
import jax.numpy as jnp
from jax.experimental import pallas as pl


def kernel(data_batch, tables):
    raise NotImplementedError("write your pallas kernel here")


# speedup vs baseline: 1.2661x; 1.2661x over previous
import functools

import jax
import jax.numpy as jnp
from jax import lax
from jax.experimental import pallas as pl
from jax.experimental.pallas import tpu as pltpu
from jax.experimental.pallas import tpu_sc as plsc

N_FIELDS = 26
VOCAB = 100000
BATCH = 16384

NW = 32
TOTAL = BATCH * N_FIELDS
CHUNK = TOTAL // NW
LANES = 16
VECS = CHUNK // LANES

_mesh = plsc.VectorSubcoreMesh(core_axis_name="c", subcore_axis_name="s")


@functools.partial(
    pl.kernel,
    mesh=_mesh,
    out_type=jax.ShapeDtypeStruct((TOTAL,), jnp.float32),
    scratch_types=[
        pltpu.VMEM((CHUNK,), jnp.int32),
        pltpu.VMEM((CHUNK,), jnp.float32),
        pltpu.SemaphoreType.DMA,
    ],
)
def _fm_gather(idx_hbm, table_hbm, out_hbm, idx_v, vals_v, sem):
    wid = lax.axis_index("s") * 2 + lax.axis_index("c")
    base = wid * CHUNK
    pltpu.sync_copy(idx_hbm.at[pl.ds(base, CHUNK)], idx_v)

    off0 = lax.iota(jnp.int32, LANES) * VOCAB
    step = jnp.int32(LANES * VOCAB)
    wrap = jnp.int32(N_FIELDS * VOCAB)

    def body(j, off):
        s = pl.ds(j * LANES, LANES)
        idx_v[s] = idx_v[s] + off
        nxt = off + step
        return jnp.where(nxt >= wrap, nxt - wrap, nxt)

    lax.fori_loop(0, VECS, body, off0)

    pltpu.async_copy(table_hbm.at[idx_v], vals_v, sem).wait()
    pltpu.sync_copy(vals_v, out_hbm.at[pl.ds(base, CHUNK)])


def kernel(data_batch, tables):
    idx = data_batch.reshape(TOTAL).astype(jnp.int32)
    table = tables.reshape(N_FIELDS * VOCAB)
    out = _fm_gather(idx, table)
    return out.reshape(BATCH, N_FIELDS)


# speedup vs baseline: 3.3902x; 2.6777x over previous
import functools

import jax
import jax.numpy as jnp
from jax import lax
from jax.experimental import pallas as pl
from jax.experimental.pallas import tpu as pltpu
from jax.experimental.pallas import tpu_sc as plsc

N_FIELDS = 26
VOCAB = 100000
PAD_VOCAB = 100096
BATCH = 16384

NW = 32
BCHUNK = BATCH // NW
CHUNK = BCHUNK * N_FIELDS
LANES = 16
KVECS = BCHUNK // LANES

_mesh = plsc.VectorSubcoreMesh(core_axis_name="c", subcore_axis_name="s")


@functools.partial(
    pl.kernel,
    mesh=_mesh,
    out_type=jax.ShapeDtypeStruct((N_FIELDS, BATCH), jnp.float32),
    scratch_types=[
        pltpu.VMEM((N_FIELDS, BCHUNK), jnp.int32),
        pltpu.VMEM((CHUNK,), jnp.int32),
        pltpu.VMEM((CHUNK,), jnp.float32),
        pltpu.SemaphoreType.DMA,
        pltpu.SemaphoreType.DMA,
    ],
)
def _fm_gather(idx_hbm, table_hbm, out_hbm, idx_v, fidx_v, vals_v, gsem, osem):
    wid = lax.axis_index("s") * 2 + lax.axis_index("c")
    col0 = wid * BCHUNK
    pltpu.sync_copy(idx_hbm.at[:, pl.ds(col0, BCHUNK)], idx_v)

    def body(j, _):
        f = j >> 5
        k = j & 31
        off = f * PAD_VOCAB
        fidx_v[pl.ds(j * LANES, LANES)] = idx_v[f, pl.ds(k * LANES, LANES)] + off
        return 0

    lax.fori_loop(0, N_FIELDS * KVECS, body, 0)

    pltpu.async_copy(table_hbm.at[fidx_v], vals_v, gsem).wait()

    copies = [
        pltpu.async_copy(
            vals_v.at[pl.ds(f * BCHUNK, BCHUNK)],
            out_hbm.at[f, pl.ds(col0, BCHUNK)],
            osem,
        )
        for f in range(N_FIELDS)
    ]
    for c in copies:
        c.wait()


def kernel(data_batch, tables):
    idx_t = data_batch.astype(jnp.int32).T
    table_flat = jnp.pad(
        tables[:, :, 0], ((0, 0), (0, PAD_VOCAB - VOCAB))
    ).reshape(N_FIELDS * PAD_VOCAB)
    out_t = _fm_gather(idx_t, table_flat)
    return out_t.T


# speedup vs baseline: 3.6183x; 1.0673x over previous
import functools

import jax
import jax.numpy as jnp
from jax import lax
from jax.experimental import pallas as pl
from jax.experimental.pallas import tpu as pltpu
from jax.experimental.pallas import tpu_sc as plsc

N_FIELDS = 26
VOCAB = 100000
BATCH = 16384

NW = 32
BCHUNK = BATCH // NW
CHUNK = BCHUNK * N_FIELDS
LANES = 16
KVECS = BCHUNK // LANES

_mesh = plsc.VectorSubcoreMesh(core_axis_name="c", subcore_axis_name="s")


@functools.partial(
    pl.kernel,
    mesh=_mesh,
    out_type=jax.ShapeDtypeStruct((N_FIELDS, BATCH), jnp.float32),
    scratch_types=[
        pltpu.VMEM((N_FIELDS, BCHUNK), jnp.int32),
        pltpu.VMEM((CHUNK,), jnp.int32),
        pltpu.VMEM((CHUNK,), jnp.float32),
        pltpu.SemaphoreType.DMA,
        pltpu.SemaphoreType.DMA,
    ],
)
def _fm_gather(idx_hbm, table_hbm, out_hbm, idx_v, fidx_v, vals_v, gsem, osem):
    wid = lax.axis_index("s") * 2 + lax.axis_index("c")
    col0 = wid * BCHUNK
    pltpu.sync_copy(idx_hbm.at[:, pl.ds(col0, BCHUNK)], idx_v)

    def body(j, _):
        f = j >> 5
        k = j & 31
        off = f * VOCAB
        fidx_v[pl.ds(j * LANES, LANES)] = idx_v[f, pl.ds(k * LANES, LANES)] + off
        return 0

    lax.fori_loop(0, N_FIELDS * KVECS, body, 0)

    pltpu.async_copy(table_hbm.at[fidx_v], vals_v, gsem).wait()

    copies = [
        pltpu.async_copy(
            vals_v.at[pl.ds(f * BCHUNK, BCHUNK)],
            out_hbm.at[f, pl.ds(col0, BCHUNK)],
            osem,
        )
        for f in range(N_FIELDS)
    ]
    for c in copies:
        c.wait()


def kernel(data_batch, tables):
    idx_t = data_batch.astype(jnp.int32).T
    table_flat = lax.optimization_barrier(tables[:, :, 0]).reshape(N_FIELDS * VOCAB)
    out_t = _fm_gather(idx_t, table_flat)
    return out_t.T


# speedup vs baseline: 3.7407x; 1.0338x over previous
import functools

import jax
import jax.numpy as jnp
from jax import lax
from jax.experimental import pallas as pl
from jax.experimental.pallas import tpu as pltpu
from jax.experimental.pallas import tpu_sc as plsc

N_FIELDS = 26
VOCAB = 100000
BATCH = 16384

NW = 32
BCHUNK = BATCH // NW

_mesh = plsc.VectorSubcoreMesh(core_axis_name="c", subcore_axis_name="s")


@functools.partial(
    pl.kernel,
    mesh=_mesh,
    out_type=jax.ShapeDtypeStruct((N_FIELDS, BATCH), jnp.float32),
    scratch_types=(
        [pltpu.VMEM((BCHUNK,), jnp.int32) for _ in range(N_FIELDS)]
        + [pltpu.VMEM((BCHUNK,), jnp.float32) for _ in range(N_FIELDS)]
        + [pltpu.SemaphoreType.DMA, pltpu.SemaphoreType.DMA, pltpu.SemaphoreType.DMA]
    ),
)
def _fm_gather(idx_hbm, table_hbm, out_hbm, *refs):
    idx_v = refs[:N_FIELDS]
    vals_v = refs[N_FIELDS:2 * N_FIELDS]
    isem, gsem, osem = refs[2 * N_FIELDS:]
    wid = lax.axis_index("s") * 2 + lax.axis_index("c")
    col0 = wid * BCHUNK

    loads = [
        pltpu.async_copy(idx_hbm.at[f, pl.ds(col0, BCHUNK)], idx_v[f], isem)
        for f in range(N_FIELDS)
    ]
    for c in loads:
        c.wait()
    gathers = [
        pltpu.async_copy(
            table_hbm.at[pl.ds(f * VOCAB, VOCAB)].at[idx_v[f]], vals_v[f], gsem
        )
        for f in range(N_FIELDS)
    ]
    for g in gathers:
        g.wait()
    stores = [
        pltpu.async_copy(vals_v[f], out_hbm.at[f, pl.ds(col0, BCHUNK)], osem)
        for f in range(N_FIELDS)
    ]
    for c in stores:
        c.wait()


def kernel(data_batch, tables):
    idx_t = data_batch.astype(jnp.int32).T
    table_flat = lax.optimization_barrier(tables[:, :, 0]).reshape(N_FIELDS * VOCAB)
    out_t = _fm_gather(idx_t, table_flat)
    return out_t.T
